# baseline (speedup 1.0000x reference)
import functools
import math

import jax
import jax.numpy as jnp
from jax import lax
from jax.experimental import pallas as pl
from jax.experimental.pallas import tpu as pltpu
from jax.experimental.pallas import tpu_sc as plsc

_N, _D, _E, _L, _O = 10000, 128, 320000, 4, 2
_EPS = 1e-5

_NC, _NS = 2, 16
_CH = 128
_NCH = 80
_EW = _CH * _NCH
_EPAD = _EW * _NC * _NS
_NPA = 10112
_RA = _NPA // _NS
_NPD = 10240
_RD = _NPD // _NS

_NCH0 = 32
_NCH1 = 128
_NBUF = 2
_PKR = _EPAD // 256
_PKT = _PKR // (_NC * _NS)

_MESH = plsc.VectorSubcoreMesh(core_axis_name="c", subcore_axis_name="s")


def _sc_agg_body(h_hbm, spk_hbm, dpk_hbm, za_hbm, outp_hbm,
                 spk_v, dpk_v, rows0, rows1, src32_0, src32_1, dst32,
                 acc_sh, sem0, sem1):
    rows = [rows0, rows1]
    sems = [sem0, sem1]
    src32 = [src32_0, src32_1]
    cid = lax.axis_index("c")
    sid = lax.axis_index("s")

    nch = jnp.where(cid == 0, _NCH0, _NCH1)
    pwords = nch * 64
    pbase = pl.multiple_of(
        jnp.where(cid == 0, sid * (_NCH0 * 64),
                  _NS * (_NCH0 * 64) + sid * (_NCH1 * 64)), 8)

    def cvt(pk_v, c, out32):
        base = c * 64
        for jj in range(4):
            w = pk_v[pl.ds(base + jj * 16, 16)]
            ev = w & 0xFFFF
            od = lax.shift_right_logical(w, 16)
            out32[pl.ds(32 * jj, 16)] = ev
            out32[pl.ds(32 * jj + 16, 16)] = od

    pltpu.sync_copy(za_hbm.at[pl.ds(sid * _RA, _RA)],
                    acc_sh.at[pl.ds(sid * _RA, _RA)])

    plsc.subcore_barrier()

    def pipeline(n_chunks):
        nw = n_chunks * 64
        pltpu.sync_copy(spk_hbm.at[pl.ds(pbase, nw)], spk_v.at[pl.ds(0, nw)])
        pltpu.sync_copy(dpk_hbm.at[pl.ds(pbase, nw)], dpk_v.at[pl.ds(0, nw)])

        for b in range(_NBUF):
            cvt(spk_v, b, src32[b])
            pltpu.async_copy(h_hbm.at[src32[b]], rows[b], sems[b])

        def group(g, carry):
            for b in range(_NBUF):
                c = g * _NBUF + b
                cvt(dpk_v, c, dst32)
                pltpu.make_async_copy(h_hbm.at[src32[b]], rows[b], sems[b]).wait()
                pltpu.sync_copy(rows[b], acc_sh.at[dst32], add=True)
                cn = c + _NBUF
                cn = jnp.where(cn >= n_chunks, cn - n_chunks, cn)
                cvt(spk_v, cn, src32[b])
                pltpu.async_copy(h_hbm.at[src32[b]], rows[b], sems[b])
            return carry

        lax.fori_loop(0, n_chunks // _NBUF, group, 0)

        for b in range(_NBUF):
            pltpu.make_async_copy(h_hbm.at[src32[b]], rows[b], sems[b]).wait()

    @pl.when(cid == 0)
    def _():
        pipeline(_NCH0)

    @pl.when(cid == 1)
    def _():
        pipeline(_NCH1)

    plsc.subcore_barrier()

    pltpu.sync_copy(acc_sh.at[pl.ds(sid * _RA, _RA)],
                    outp_hbm.at[cid, pl.ds(sid * _RA, _RA)])


_sc_agg = functools.partial(
    pl.kernel,
    out_type=jax.ShapeDtypeStruct((_NC, _NPA, _D), jnp.float32),
    mesh=_MESH,
    scratch_types=[
        pltpu.VMEM((_NCH1 * 64,), jnp.int32),
        pltpu.VMEM((_NCH1 * 64,), jnp.int32),
        pltpu.VMEM((_CH, _D), jnp.float32),
        pltpu.VMEM((_CH, _D), jnp.float32),
        pltpu.VMEM((_CH,), jnp.int32),
        pltpu.VMEM((_CH,), jnp.int32),
        pltpu.VMEM((_CH,), jnp.int32),
        pltpu.VMEM_SHARED((_NPA, _D), jnp.float32),
        pltpu.SemaphoreType.DMA,
        pltpu.SemaphoreType.DMA,
    ],
)(_sc_agg_body)


def _sc_deg_body(dst_hbm, zd_hbm, outd_hbm, dst_v, ones_v, deg_sh):
    cid = lax.axis_index("c")
    sid = lax.axis_index("s")
    wid = cid * _NS + sid

    pltpu.sync_copy(zd_hbm.at[pl.ds(sid * _RD, _RD)],
                    deg_sh.at[pl.ds(sid * _RD, _RD)])

    ebase = pl.multiple_of(wid * _NCH, 8)
    pltpu.sync_copy(dst_hbm.at[pl.ds(ebase, _NCH)], dst_v)
    for j in range(_CH // 16):
        ones_v[pl.ds(j * 16, 16)] = jnp.full((16,), 1.0, jnp.float32)

    plsc.subcore_barrier()

    def chunk(c, carry):
        pltpu.sync_copy(ones_v, deg_sh.at[dst_v.at[c]], add=True)
        return carry

    lax.fori_loop(0, _NCH, chunk, 0)

    plsc.subcore_barrier()

    pltpu.sync_copy(deg_sh.at[pl.ds(sid * _RD, _RD)],
                    outd_hbm.at[cid, pl.ds(sid * _RD, _RD)])


_sc_deg = functools.partial(
    pl.kernel,
    out_type=jax.ShapeDtypeStruct((_NC, _NPD), jnp.float32),
    mesh=_MESH,
    scratch_types=[
        pltpu.VMEM((_NCH, _CH), jnp.int32),
        pltpu.VMEM((_CH,), jnp.float32),
        pltpu.VMEM_SHARED((_NPD,), jnp.float32),
    ],
)(_sc_deg_body)


_R = 1000
_G = _N // _R


def _proj_body(x_ref, w_ref, b_ref, o_ref):
    o_ref[...] = jnp.maximum(
        jnp.dot(x_ref[...], w_ref[...], preferred_element_type=jnp.float32)
        + b_ref[...], 0.0)


def _proj(x, w, b):
    return pl.pallas_call(
        _proj_body,
        grid=(_G,),
        in_specs=[
            pl.BlockSpec((_R, _D), lambda i: (i, 0)),
            pl.BlockSpec((_D, _D), lambda i: (0, 0)),
            pl.BlockSpec((1, _D), lambda i: (0, 0)),
        ],
        out_specs=pl.BlockSpec((_R, _D), lambda i: (i, 0)),
        out_shape=jax.ShapeDtypeStruct((_N, _D), jnp.float32),
    )(x, w, b)


def _layer_core(p_ref, d_ref, h_ref, wl_ref, wr_ref, sc_ref, sh_ref):
    s = p_ref[0] + p_ref[1]
    t = jnp.dot(s, wl_ref[...], preferred_element_type=jnp.float32)
    rec = 1.0 / jnp.maximum(d_ref[0] + d_ref[1], 1.0)
    u = jnp.dot(h_ref[...], wr_ref[...], preferred_element_type=jnp.float32)
    v = (t * rec + u) * sc_ref[...] + sh_ref[...]
    return jnp.maximum(v, 0.0) + h_ref[...]


def _layer_body(p_ref, d_ref, h_ref, wl_ref, wr_ref, sc_ref, sh_ref, o_ref):
    o_ref[...] = _layer_core(p_ref, d_ref, h_ref, wl_ref, wr_ref, sc_ref, sh_ref)


def _final_body(p_ref, d_ref, h_ref, wl_ref, wr_ref, sc_ref, sh_ref,
                wc_ref, bc_ref, o_ref):
    hh = _layer_core(p_ref, d_ref, h_ref, wl_ref, wr_ref, sc_ref, sh_ref)
    logits = jnp.dot(hh, wc_ref[...], preferred_element_type=jnp.float32) + bc_ref[...]
    col = lax.broadcasted_iota(jnp.int32, logits.shape, 1)
    masked = jnp.where(col < _O, logits, -jnp.inf)
    m = jnp.max(masked, axis=1, keepdims=True)
    lse = m + jnp.log(jnp.sum(jnp.exp(masked - m), axis=1, keepdims=True))
    o_ref[...] = logits - lse


def _p_spec():
    return pl.BlockSpec((_NC, _R, _D), lambda i: (0, i, 0))


def _d_spec():
    return pl.BlockSpec((_NC, _R, 1), lambda i: (0, i, 0))


def _layer(P, Dg3, h, wl, wr, sc, sh):
    return pl.pallas_call(
        _layer_body,
        grid=(_G,),
        in_specs=[
            _p_spec(), _d_spec(),
            pl.BlockSpec((_R, _D), lambda i: (i, 0)),
            pl.BlockSpec((_D, _D), lambda i: (0, 0)),
            pl.BlockSpec((_D, _D), lambda i: (0, 0)),
            pl.BlockSpec((1, _D), lambda i: (0, 0)),
            pl.BlockSpec((1, _D), lambda i: (0, 0)),
        ],
        out_specs=pl.BlockSpec((_R, _D), lambda i: (i, 0)),
        out_shape=jax.ShapeDtypeStruct((_N, _D), jnp.float32),
    )(P, Dg3, h, wl, wr, sc, sh)


def _final(P, Dg3, h, wl, wr, sc, sh, wc, bc):
    return pl.pallas_call(
        _final_body,
        grid=(_G,),
        in_specs=[
            _p_spec(), _d_spec(),
            pl.BlockSpec((_R, _D), lambda i: (i, 0)),
            pl.BlockSpec((_D, _D), lambda i: (0, 0)),
            pl.BlockSpec((_D, _D), lambda i: (0, 0)),
            pl.BlockSpec((1, _D), lambda i: (0, 0)),
            pl.BlockSpec((1, _D), lambda i: (0, 0)),
            pl.BlockSpec((_D, _D), lambda i: (0, 0)),
            pl.BlockSpec((1, _D), lambda i: (0, 0)),
        ],
        out_specs=pl.BlockSpec((_R, _D), lambda i: (i, 0)),
        out_shape=jax.ShapeDtypeStruct((_N, _D), jnp.float32),
    )(P, Dg3, h, wl, wr, sc, sh, wc, bc)


def kernel(x, edge_index, W_in, b_in, Wl, bl, Wr, gamma, beta, W_cls, b_cls):
    f32 = jnp.float32
    src = edge_index[0]
    dst = edge_index[1]
    npad = _EPAD - _E
    sink = _N + (jnp.arange(npad, dtype=jnp.int32) % (_NPA - _N))
    srcp = jnp.concatenate([src, jnp.zeros((npad,), jnp.int32)])
    dstp = jnp.concatenate([dst, sink])
    dst2 = dstp.reshape(_EPAD // _CH, _CH)

    def pack16(v):
        v2 = v.reshape(_EPAD // _CH, 2, _CH // 2)
        return (v2[:, 0] | (v2[:, 1] << 16)).reshape(-1)

    spk = pack16(srcp)
    dpk = pack16(dstp)
    za = jnp.zeros((_NPA, _D), f32)
    zd = jnp.zeros((_NPD,), f32)

    inv = f32(1.0 / math.sqrt(1.0 + _EPS))
    scale = gamma * inv
    shift = beta + bl * scale
    wc = jnp.zeros((_D, _D), f32).at[:, :_O].set(W_cls)
    bc = jnp.zeros((1, _D), f32).at[0, :_O].set(b_cls)

    Dg = _sc_deg(dst2, zd)
    Dg3 = Dg.reshape(_NC, _NPD, 1)

    h = _proj(x, W_in, b_in.reshape(1, _D))
    for i in range(_L - 1):
        P = _sc_agg(h, spk, dpk, za)
        h = _layer(P, Dg3, h, Wl[i], Wr[i],
                   scale[i].reshape(1, _D), shift[i].reshape(1, _D))
    P = _sc_agg(h, spk, dpk, za)
    full = _final(P, Dg3, h, Wl[_L - 1], Wr[_L - 1],
                  scale[_L - 1].reshape(1, _D), shift[_L - 1].reshape(1, _D),
                  wc, bc)
    return full[:, :_O]


# speedup vs baseline: 1.0854x; 1.0854x over previous
import functools
import math

import jax
import jax.numpy as jnp
from jax import lax
from jax.experimental import pallas as pl
from jax.experimental.pallas import tpu as pltpu
from jax.experimental.pallas import tpu_sc as plsc

_N, _D, _E, _L, _O = 10000, 128, 320000, 4, 2
_EPS = 1e-5

_NC, _NS = 2, 16
_CH = 128
_NCH = 80
_EW = _CH * _NCH
_EPAD = _EW * _NC * _NS
_NPA = 10112
_RA = _NPA // _NS
_NPD = 10240
_RD = _NPD // _NS

_NCH0 = 128
_NCH1 = 32
_NBUF = 2
_NCHMAX = max(_NCH0, _NCH1)
_PKR = _EPAD // 256
_PKT = _PKR // (_NC * _NS)

_MESH = plsc.VectorSubcoreMesh(core_axis_name="c", subcore_axis_name="s")


def _sc_agg_body(h_hbm, spk_hbm, dpk_hbm, za_hbm, outp_hbm,
                 spk_v, dpk_v, rows0, rows1, src32_0, src32_1, dst32,
                 acc_sh, sem0, sem1):
    rows = [rows0, rows1]
    sems = [sem0, sem1]
    src32 = [src32_0, src32_1]
    cid = lax.axis_index("c")
    sid = lax.axis_index("s")

    nch = jnp.where(cid == 0, _NCH0, _NCH1)
    pwords = nch * 64
    pbase = pl.multiple_of(
        jnp.where(cid == 0, sid * (_NCH0 * 64),
                  _NS * (_NCH0 * 64) + sid * (_NCH1 * 64)), 8)

    def cvt(pk_v, c, out32):
        base = c * 64
        for jj in range(4):
            w = pk_v[pl.ds(base + jj * 16, 16)]
            ev = w & 0xFFFF
            od = lax.shift_right_logical(w, 16)
            out32[pl.ds(32 * jj, 16)] = ev
            out32[pl.ds(32 * jj + 16, 16)] = od

    pltpu.sync_copy(za_hbm.at[pl.ds(sid * _RA, _RA)],
                    acc_sh.at[pl.ds(sid * _RA, _RA)])

    plsc.subcore_barrier()

    def pipeline(n_chunks):
        nw = n_chunks * 64
        pltpu.sync_copy(spk_hbm.at[pl.ds(pbase, nw)], spk_v.at[pl.ds(0, nw)])
        pltpu.sync_copy(dpk_hbm.at[pl.ds(pbase, nw)], dpk_v.at[pl.ds(0, nw)])

        for b in range(_NBUF):
            cvt(spk_v, b, src32[b])
            pltpu.async_copy(h_hbm.at[src32[b]], rows[b], sems[b])

        def group(g, carry):
            for b in range(_NBUF):
                c = g * _NBUF + b
                cvt(dpk_v, c, dst32)
                pltpu.make_async_copy(h_hbm.at[src32[b]], rows[b], sems[b]).wait()
                pltpu.sync_copy(rows[b], acc_sh.at[dst32], add=True)
                cn = c + _NBUF
                cn = jnp.where(cn >= n_chunks, cn - n_chunks, cn)
                cvt(spk_v, cn, src32[b])
                pltpu.async_copy(h_hbm.at[src32[b]], rows[b], sems[b])
            return carry

        lax.fori_loop(0, n_chunks // _NBUF, group, 0)

        for b in range(_NBUF):
            pltpu.make_async_copy(h_hbm.at[src32[b]], rows[b], sems[b]).wait()

    @pl.when(cid == 0)
    def _():
        pipeline(_NCH0)

    @pl.when(cid == 1)
    def _():
        pipeline(_NCH1)

    plsc.subcore_barrier()

    pltpu.sync_copy(acc_sh.at[pl.ds(sid * _RA, _RA)],
                    outp_hbm.at[cid, pl.ds(sid * _RA, _RA)])


_sc_agg = functools.partial(
    pl.kernel,
    out_type=jax.ShapeDtypeStruct((_NC, _NPA, _D), jnp.float32),
    mesh=_MESH,
    scratch_types=[
        pltpu.VMEM((_NCHMAX * 64,), jnp.int32),
        pltpu.VMEM((_NCHMAX * 64,), jnp.int32),
        pltpu.VMEM((_CH, _D), jnp.float32),
        pltpu.VMEM((_CH, _D), jnp.float32),
        pltpu.VMEM((_CH,), jnp.int32),
        pltpu.VMEM((_CH,), jnp.int32),
        pltpu.VMEM((_CH,), jnp.int32),
        pltpu.VMEM_SHARED((_NPA, _D), jnp.float32),
        pltpu.SemaphoreType.DMA,
        pltpu.SemaphoreType.DMA,
    ],
)(_sc_agg_body)


def _sc_deg_body(dst_hbm, zd_hbm, outd_hbm, dst_v, ones_v, deg_sh):
    cid = lax.axis_index("c")
    sid = lax.axis_index("s")
    wid = cid * _NS + sid

    pltpu.sync_copy(zd_hbm.at[pl.ds(sid * _RD, _RD)],
                    deg_sh.at[pl.ds(sid * _RD, _RD)])

    ebase = pl.multiple_of(wid * _NCH, 8)
    pltpu.sync_copy(dst_hbm.at[pl.ds(ebase, _NCH)], dst_v)
    for j in range(_CH // 16):
        ones_v[pl.ds(j * 16, 16)] = jnp.full((16,), 1.0, jnp.float32)

    plsc.subcore_barrier()

    def chunk(c, carry):
        pltpu.sync_copy(ones_v, deg_sh.at[dst_v.at[c]], add=True)
        return carry

    lax.fori_loop(0, _NCH, chunk, 0)

    plsc.subcore_barrier()

    pltpu.sync_copy(deg_sh.at[pl.ds(sid * _RD, _RD)],
                    outd_hbm.at[cid, pl.ds(sid * _RD, _RD)])


_sc_deg = functools.partial(
    pl.kernel,
    out_type=jax.ShapeDtypeStruct((_NC, _NPD), jnp.float32),
    mesh=_MESH,
    scratch_types=[
        pltpu.VMEM((_NCH, _CH), jnp.int32),
        pltpu.VMEM((_CH,), jnp.float32),
        pltpu.VMEM_SHARED((_NPD,), jnp.float32),
    ],
)(_sc_deg_body)


_R = 1000
_G = _N // _R


def _proj_body(x_ref, w_ref, b_ref, o_ref):
    o_ref[...] = jnp.maximum(
        jnp.dot(x_ref[...], w_ref[...], preferred_element_type=jnp.float32)
        + b_ref[...], 0.0)


def _proj(x, w, b):
    return pl.pallas_call(
        _proj_body,
        grid=(_G,),
        in_specs=[
            pl.BlockSpec((_R, _D), lambda i: (i, 0)),
            pl.BlockSpec((_D, _D), lambda i: (0, 0)),
            pl.BlockSpec((1, _D), lambda i: (0, 0)),
        ],
        out_specs=pl.BlockSpec((_R, _D), lambda i: (i, 0)),
        out_shape=jax.ShapeDtypeStruct((_N, _D), jnp.float32),
    )(x, w, b)


def _layer_core(p_ref, d_ref, h_ref, wl_ref, wr_ref, sc_ref, sh_ref):
    s = p_ref[0] + p_ref[1]
    t = jnp.dot(s, wl_ref[...], preferred_element_type=jnp.float32)
    rec = 1.0 / jnp.maximum(d_ref[0] + d_ref[1], 1.0)
    u = jnp.dot(h_ref[...], wr_ref[...], preferred_element_type=jnp.float32)
    v = (t * rec + u) * sc_ref[...] + sh_ref[...]
    return jnp.maximum(v, 0.0) + h_ref[...]


def _layer_body(p_ref, d_ref, h_ref, wl_ref, wr_ref, sc_ref, sh_ref, o_ref):
    o_ref[...] = _layer_core(p_ref, d_ref, h_ref, wl_ref, wr_ref, sc_ref, sh_ref)


def _final_body(p_ref, d_ref, h_ref, wl_ref, wr_ref, sc_ref, sh_ref,
                wc_ref, bc_ref, o_ref):
    hh = _layer_core(p_ref, d_ref, h_ref, wl_ref, wr_ref, sc_ref, sh_ref)
    logits = jnp.dot(hh, wc_ref[...], preferred_element_type=jnp.float32) + bc_ref[...]
    col = lax.broadcasted_iota(jnp.int32, logits.shape, 1)
    masked = jnp.where(col < _O, logits, -jnp.inf)
    m = jnp.max(masked, axis=1, keepdims=True)
    lse = m + jnp.log(jnp.sum(jnp.exp(masked - m), axis=1, keepdims=True))
    o_ref[...] = logits - lse


def _p_spec():
    return pl.BlockSpec((_NC, _R, _D), lambda i: (0, i, 0))


def _d_spec():
    return pl.BlockSpec((_NC, _R, 1), lambda i: (0, i, 0))


def _layer(P, Dg3, h, wl, wr, sc, sh):
    return pl.pallas_call(
        _layer_body,
        grid=(_G,),
        in_specs=[
            _p_spec(), _d_spec(),
            pl.BlockSpec((_R, _D), lambda i: (i, 0)),
            pl.BlockSpec((_D, _D), lambda i: (0, 0)),
            pl.BlockSpec((_D, _D), lambda i: (0, 0)),
            pl.BlockSpec((1, _D), lambda i: (0, 0)),
            pl.BlockSpec((1, _D), lambda i: (0, 0)),
        ],
        out_specs=pl.BlockSpec((_R, _D), lambda i: (i, 0)),
        out_shape=jax.ShapeDtypeStruct((_N, _D), jnp.float32),
    )(P, Dg3, h, wl, wr, sc, sh)


def _final(P, Dg3, h, wl, wr, sc, sh, wc, bc):
    return pl.pallas_call(
        _final_body,
        grid=(_G,),
        in_specs=[
            _p_spec(), _d_spec(),
            pl.BlockSpec((_R, _D), lambda i: (i, 0)),
            pl.BlockSpec((_D, _D), lambda i: (0, 0)),
            pl.BlockSpec((_D, _D), lambda i: (0, 0)),
            pl.BlockSpec((1, _D), lambda i: (0, 0)),
            pl.BlockSpec((1, _D), lambda i: (0, 0)),
            pl.BlockSpec((_D, _D), lambda i: (0, 0)),
            pl.BlockSpec((1, _D), lambda i: (0, 0)),
        ],
        out_specs=pl.BlockSpec((_R, _D), lambda i: (i, 0)),
        out_shape=jax.ShapeDtypeStruct((_N, _D), jnp.float32),
    )(P, Dg3, h, wl, wr, sc, sh, wc, bc)


def kernel(x, edge_index, W_in, b_in, Wl, bl, Wr, gamma, beta, W_cls, b_cls):
    f32 = jnp.float32
    src = edge_index[0]
    dst = edge_index[1]
    npad = _EPAD - _E
    sink = _N + (jnp.arange(npad, dtype=jnp.int32) % (_NPA - _N))
    srcp = jnp.concatenate([src, jnp.zeros((npad,), jnp.int32)])
    dstp = jnp.concatenate([dst, sink])
    dst2 = dstp.reshape(_EPAD // _CH, _CH)

    def pack16(v):
        v2 = v.reshape(_EPAD // _CH, 2, _CH // 2)
        return (v2[:, 0] | (v2[:, 1] << 16)).reshape(-1)

    spk = pack16(srcp)
    dpk = pack16(dstp)
    za = jnp.zeros((_NPA, _D), f32)
    zd = jnp.zeros((_NPD,), f32)

    inv = f32(1.0 / math.sqrt(1.0 + _EPS))
    scale = gamma * inv
    shift = beta + bl * scale
    wc = jnp.zeros((_D, _D), f32).at[:, :_O].set(W_cls)
    bc = jnp.zeros((1, _D), f32).at[0, :_O].set(b_cls)

    Dg = _sc_deg(dst2, zd)
    Dg3 = Dg.reshape(_NC, _NPD, 1)

    h = _proj(x, W_in, b_in.reshape(1, _D))
    for i in range(_L - 1):
        P = _sc_agg(h, spk, dpk, za)
        h = _layer(P, Dg3, h, Wl[i], Wr[i],
                   scale[i].reshape(1, _D), shift[i].reshape(1, _D))
    P = _sc_agg(h, spk, dpk, za)
    full = _final(P, Dg3, h, Wl[_L - 1], Wr[_L - 1],
                  scale[_L - 1].reshape(1, _D), shift[_L - 1].reshape(1, _D),
                  wc, bc)
    return full[:, :_O]
